# baseline (speedup 1.0000x reference)
import functools

import jax
import jax.numpy as jnp
from jax import lax
from jax.experimental import pallas as pl
from jax.experimental.pallas import tpu as pltpu
from jax.experimental.pallas import tpu_sc as plsc

EPS = 0.1


def _pick_bm(n, target):
    for bm in range(min(n, target), 0, -1):
        if n % bm == 0 and (bm % 8 == 0 or bm == n):
            return bm
    return n


def _sc_gather_x(idx_hbm, x_hbm, out_x, idx_v, xrows_v, sem, *, b_per_w):
    nc = plsc.get_sparse_core_info().num_cores
    wid = lax.axis_index("s") * nc + lax.axis_index("c")
    base = wid * b_per_w
    pltpu.sync_copy(idx_hbm.at[pl.ds(base, b_per_w)], idx_v)
    pltpu.async_copy(x_hbm.at[idx_v], xrows_v, sem).wait()
    pltpu.sync_copy(xrows_v, out_x.at[pl.ds(base, b_per_w)])


def _sc_gather_adj(idx_hbm, adj_hbm, out_adj, idx_v, rows_v, sem,
                   *, b_per_w, chunk, ncut):
    nc = plsc.get_sparse_core_info().num_cores
    wid = lax.axis_index("s") * nc + lax.axis_index("c")
    base = wid * b_per_w
    pltpu.sync_copy(idx_hbm.at[pl.ds(base, b_per_w)], idx_v)
    for c in range(b_per_w // chunk):
        idx_c = idx_v.at[pl.ds(c * chunk, chunk)]
        pltpu.async_copy(adj_hbm.at[idx_c, pl.ds(0, ncut)], rows_v, sem).wait()
        pltpu.sync_copy(rows_v, out_adj.at[pl.ds(base + c * chunk, chunk)])


def _delta_body(x_sel_ref, noise_ref, w1_ref, out_ref):
    noise = noise_ref[...]
    nrm = jnp.sqrt(jnp.sum(noise * noise, axis=1, keepdims=True)) + 1e-12
    delta = jnp.sign(x_sel_ref[...]) * (noise / nrm) * EPS
    out_ref[...] = jnp.dot(delta, w1_ref[...], preferred_element_type=jnp.float32)


def _b1_body(nn_ref, x_ref, w1_ref, dw1_ref, out_ref, *, bm, nn_count):
    blk = pl.program_id(0)
    rows = blk * bm + jax.lax.broadcasted_iota(jnp.int32, (bm, nn_count), 0)
    onehot = (rows == nn_ref[...]).astype(jnp.bfloat16)
    d1 = jnp.dot(onehot, dw1_ref[...].astype(jnp.bfloat16),
                 preferred_element_type=jnp.float32)
    p = jnp.dot(x_ref[...].astype(jnp.bfloat16),
                w1_ref[...].astype(jnp.bfloat16),
                preferred_element_type=jnp.float32)
    out_ref[...] = jnp.concatenate([p + d1, p], axis=1).astype(jnp.bfloat16)


def _pass1_body(adj_ref, b1_ref, w2_ref, out_ref, *, h1):
    adj_bf = adj_ref[...].astype(jnp.bfloat16)
    h = jnp.maximum(
        jnp.dot(adj_bf, b1_ref[...], preferred_element_type=jnp.float32), 0.0)
    hb = h.astype(jnp.bfloat16)
    w2 = w2_ref[...]
    b2a = jnp.dot(hb[:, :h1], w2, preferred_element_type=jnp.float32)
    b2b = jnp.dot(hb[:, h1:], w2, preferred_element_type=jnp.float32)
    out_ref[...] = jnp.concatenate([b2a, b2b], axis=1).astype(jnp.bfloat16)


def _pass2_body(adj_ref, b2_ref, we2d_ref, z_ref, emb_ref, rep_ref, *, h2):
    adj_bf = adj_ref[...].astype(jnp.bfloat16)
    ze = jnp.dot(adj_bf, b2_ref[...], preferred_element_type=jnp.float32)
    z = ze[:, :h2]
    z_ref[...] = z
    emb_ref[...] = ze[:, h2:]
    rep_ref[...] = jnp.dot(z, we2d_ref[...], preferred_element_type=jnp.float32)


def _dec_body(idx_ref, adj_ref, rep_ref, wdec_ref, out_ref, buf, sem, *, bm5):
    i = pl.program_id(0)
    for j in range(bm5):
        pltpu.make_async_copy(
            adj_ref.at[pl.ds(idx_ref[i * bm5 + j], 1), :],
            buf.at[pl.ds(j, 1), :], sem).start()
    for j in range(bm5):
        pltpu.make_async_copy(
            adj_ref.at[pl.ds(idx_ref[i * bm5 + j], 1), :],
            buf.at[pl.ds(j, 1), :], sem).wait()
    rep_bf = rep_ref[...].astype(jnp.bfloat16)
    t = jnp.dot(buf[...].astype(jnp.bfloat16), rep_bf,
                preferred_element_type=jnp.float32)
    out_ref[...] = jnp.dot(t.astype(jnp.bfloat16),
                           wdec_ref[...].astype(jnp.bfloat16),
                           preferred_element_type=jnp.float32)


def kernel(adj, x, noise_nodes, W_enc1, W_enc2, W_e2d, W_dec):
    n, f_in = x.shape
    h1 = W_enc1.shape[1]
    h2 = W_enc2.shape[1]
    nn_count = noise_nodes.shape[0]
    idx = noise_nodes.astype(jnp.int32)
    nn2d = idx.reshape(1, nn_count)
    with jax.ensure_compile_time_eval():
        noise = jax.random.uniform(
            jax.random.key(42), (nn_count, f_in), dtype=x.dtype)
    ncut = (n // 128) * 128

    info = plsc.get_sparse_core_info()
    nw = info.num_cores * info.num_subcores
    npad = ((nn_count + 8 * nw - 1) // (8 * nw)) * (8 * nw)
    idx_pad = jnp.pad(idx, (0, npad - nn_count))
    b_per_w = npad // nw
    chunk = 8
    mesh = plsc.VectorSubcoreMesh(core_axis_name="c", subcore_axis_name="s")
    x_sel_p = pl.kernel(
        functools.partial(_sc_gather_x, b_per_w=b_per_w),
        out_type=jax.ShapeDtypeStruct((npad, f_in), jnp.float32),
        mesh=mesh,
        scratch_types=[
            pltpu.VMEM((b_per_w,), jnp.int32),
            pltpu.VMEM((b_per_w, f_in), jnp.float32),
            pltpu.SemaphoreType.DMA,
        ],
    )(idx_pad, x)
    x_sel = x_sel_p[:nn_count]

    dw1 = pl.pallas_call(
        _delta_body,
        out_shape=jax.ShapeDtypeStruct((nn_count, h1), jnp.float32),
    )(x_sel, noise, W_enc1)

    bm = _pick_bm(n, 200)
    grid_n = n // bm
    b1 = pl.pallas_call(
        functools.partial(_b1_body, bm=bm, nn_count=nn_count),
        grid=(grid_n,),
        in_specs=[
            pl.BlockSpec((1, nn_count), lambda i: (0, 0)),
            pl.BlockSpec((bm, f_in), lambda i: (i, 0)),
            pl.BlockSpec((f_in, h1), lambda i: (0, 0)),
            pl.BlockSpec((nn_count, h1), lambda i: (0, 0)),
        ],
        out_specs=pl.BlockSpec((bm, 2 * h1), lambda i: (i, 0)),
        out_shape=jax.ShapeDtypeStruct((n, 2 * h1), jnp.bfloat16),
    )(nn2d, x, W_enc1, dw1)

    w2_bf = W_enc2.astype(jnp.bfloat16)
    b2 = pl.pallas_call(
        functools.partial(_pass1_body, h1=h1),
        grid=(grid_n,),
        in_specs=[
            pl.BlockSpec((bm, n), lambda i: (i, 0)),
            pl.BlockSpec((n, 2 * h1), lambda i: (0, 0)),
            pl.BlockSpec((h1, h2), lambda i: (0, 0)),
        ],
        out_specs=pl.BlockSpec((bm, 2 * h2), lambda i: (i, 0)),
        out_shape=jax.ShapeDtypeStruct((n, 2 * h2), jnp.bfloat16),
        compiler_params=pltpu.CompilerParams(
            dimension_semantics=("arbitrary",)),
    )(adj, b1, w2_bf)

    z, emb, rep = pl.pallas_call(
        functools.partial(_pass2_body, h2=h2),
        grid=(grid_n,),
        in_specs=[
            pl.BlockSpec((bm, n), lambda i: (i, 0)),
            pl.BlockSpec((n, 2 * h2), lambda i: (0, 0)),
            pl.BlockSpec((h2, h2), lambda i: (0, 0)),
        ],
        out_specs=[
            pl.BlockSpec((bm, h2), lambda i: (i, 0)),
            pl.BlockSpec((bm, h2), lambda i: (i, 0)),
            pl.BlockSpec((bm, h2), lambda i: (i, 0)),
        ],
        out_shape=[
            jax.ShapeDtypeStruct((n, h2), jnp.float32),
            jax.ShapeDtypeStruct((n, h2), jnp.float32),
            jax.ShapeDtypeStruct((n, h2), jnp.float32),
        ],
        compiler_params=pltpu.CompilerParams(
            dimension_semantics=("arbitrary",)),
    )(adj, b2, W_e2d)

    bm5 = _pick_bm(nn_count, 200)
    x_rec = pl.pallas_call(
        functools.partial(_dec_body, bm5=bm5),
        grid_spec=pltpu.PrefetchScalarGridSpec(
            num_scalar_prefetch=1,
            grid=(nn_count // bm5,),
            in_specs=[
                pl.BlockSpec(memory_space=pl.ANY),
                pl.BlockSpec((n, h2), lambda i, idx_ref: (0, 0)),
                pl.BlockSpec((h2, f_in), lambda i, idx_ref: (0, 0)),
            ],
            out_specs=pl.BlockSpec((bm5, f_in), lambda i, idx_ref: (i, 0)),
            scratch_shapes=[
                pltpu.VMEM((bm5, n), jnp.float32),
                pltpu.SemaphoreType.DMA,
            ],
        ),
        out_shape=jax.ShapeDtypeStruct((nn_count, f_in), jnp.float32),
    )(idx, adj, rep, W_dec)

    return (x_sel, x_rec, emb, rep, z)


# speedup vs baseline: 1.0979x; 1.0979x over previous
import functools

import jax
import jax.numpy as jnp
from jax import lax
from jax.experimental import pallas as pl
from jax.experimental.pallas import tpu as pltpu
from jax.experimental.pallas import tpu_sc as plsc

EPS = 0.1


def _pick_bm(n, target):
    for bm in range(min(n, target), 0, -1):
        if n % bm == 0 and (bm % 8 == 0 or bm == n):
            return bm
    return n


def _sc_gather_x(idx_hbm, x_hbm, out_x, idx_v, xrows_v, sem, *, b_per_w):
    nc = plsc.get_sparse_core_info().num_cores
    wid = lax.axis_index("s") * nc + lax.axis_index("c")
    base = wid * b_per_w
    pltpu.sync_copy(idx_hbm.at[pl.ds(base, b_per_w)], idx_v)
    pltpu.async_copy(x_hbm.at[idx_v], xrows_v, sem).wait()
    pltpu.sync_copy(xrows_v, out_x.at[pl.ds(base, b_per_w)])


def _sc_gather_adj(idx_hbm, adj_hbm, out_adj, idx_v, rows_v, sem,
                   *, b_per_w, chunk, ncut):
    nc = plsc.get_sparse_core_info().num_cores
    wid = lax.axis_index("s") * nc + lax.axis_index("c")
    base = wid * b_per_w
    pltpu.sync_copy(idx_hbm.at[pl.ds(base, b_per_w)], idx_v)
    for c in range(b_per_w // chunk):
        idx_c = idx_v.at[pl.ds(c * chunk, chunk)]
        pltpu.async_copy(adj_hbm.at[idx_c, pl.ds(0, ncut)], rows_v, sem).wait()
        pltpu.sync_copy(rows_v, out_adj.at[pl.ds(base + c * chunk, chunk)])


def _delta_body(x_sel_ref, noise_ref, w1_ref, out_ref):
    noise = noise_ref[...]
    nrm = jnp.sqrt(jnp.sum(noise * noise, axis=1, keepdims=True)) + 1e-12
    delta = jnp.sign(x_sel_ref[...]) * (noise / nrm) * EPS
    out_ref[...] = jnp.dot(delta, w1_ref[...], preferred_element_type=jnp.float32)


def _pass1_body(nn_ref, xsel_ref, noisen_ref, x_ref, w1_ref, w2_ref, adj_ref,
                out_ref, b1_scr, *, h1, bm, n, nn_count):
    i = pl.program_id(0)

    @pl.when(i == 0)
    def _():
        w1b = w1_ref[...].astype(jnp.bfloat16)
        delta = jnp.sign(xsel_ref[...]) * noisen_ref[...]
        dw1 = jnp.dot(delta.astype(jnp.bfloat16), w1b,
                      preferred_element_type=jnp.float32).astype(jnp.bfloat16)
        for c in range(n // bm):
            rows = c * bm + jax.lax.broadcasted_iota(
                jnp.int32, (bm, nn_count), 0)
            oh = (rows == nn_ref[...]).astype(jnp.bfloat16)
            d1 = jnp.dot(oh, dw1, preferred_element_type=jnp.float32)
            p = jnp.dot(x_ref[pl.ds(c * bm, bm), :].astype(jnp.bfloat16), w1b,
                        preferred_element_type=jnp.float32)
            b1_scr[pl.ds(c * bm, bm), :] = jnp.concatenate(
                [p + d1, p], axis=1).astype(jnp.bfloat16)

    adj_bf = adj_ref[...].astype(jnp.bfloat16)
    h = jnp.maximum(
        jnp.dot(adj_bf, b1_scr[...], preferred_element_type=jnp.float32), 0.0)
    hb = h.astype(jnp.bfloat16)
    w2 = w2_ref[...]
    b2a = jnp.dot(hb[:, :h1], w2, preferred_element_type=jnp.float32)
    b2b = jnp.dot(hb[:, h1:], w2, preferred_element_type=jnp.float32)
    out_ref[...] = jnp.concatenate([b2a, b2b], axis=1).astype(jnp.bfloat16)


def _pass2_body(adj_ref, b2_ref, we2d_ref, z_ref, emb_ref, rep_ref, *, h2):
    adj_bf = adj_ref[...].astype(jnp.bfloat16)
    ze = jnp.dot(adj_bf, b2_ref[...], preferred_element_type=jnp.float32)
    z = ze[:, :h2]
    z_ref[...] = z
    emb_ref[...] = ze[:, h2:]
    rep_ref[...] = jnp.dot(z, we2d_ref[...], preferred_element_type=jnp.float32)


def _dec_body(idx_ref, adj_ref, rep_ref, wdec_ref, out_ref, buf, sem, *, bm5):
    i = pl.program_id(0)
    for j in range(bm5):
        pltpu.make_async_copy(
            adj_ref.at[pl.ds(idx_ref[i * bm5 + j], 1), :],
            buf.at[pl.ds(j, 1), :], sem).start()
    for j in range(bm5):
        pltpu.make_async_copy(
            adj_ref.at[pl.ds(idx_ref[i * bm5 + j], 1), :],
            buf.at[pl.ds(j, 1), :], sem).wait()
    rep_bf = rep_ref[...].astype(jnp.bfloat16)
    t = jnp.dot(buf[...].astype(jnp.bfloat16), rep_bf,
                preferred_element_type=jnp.float32)
    out_ref[...] = jnp.dot(t.astype(jnp.bfloat16),
                           wdec_ref[...].astype(jnp.bfloat16),
                           preferred_element_type=jnp.float32)


def kernel(adj, x, noise_nodes, W_enc1, W_enc2, W_e2d, W_dec):
    n, f_in = x.shape
    h1 = W_enc1.shape[1]
    h2 = W_enc2.shape[1]
    nn_count = noise_nodes.shape[0]
    idx = noise_nodes.astype(jnp.int32)
    nn2d = idx.reshape(1, nn_count)
    noise = jax.random.uniform(
        jax.random.key(42), (nn_count, f_in), dtype=x.dtype)
    noise_n = noise / (jnp.linalg.norm(
        noise, axis=-1, keepdims=True) + 1e-12) * EPS
    ncut = (n // 128) * 128

    info = plsc.get_sparse_core_info()
    nw = info.num_cores * info.num_subcores
    npad = ((nn_count + 8 * nw - 1) // (8 * nw)) * (8 * nw)
    idx_pad = jnp.pad(idx, (0, npad - nn_count))
    b_per_w = npad // nw
    chunk = 8
    mesh = plsc.VectorSubcoreMesh(core_axis_name="c", subcore_axis_name="s")
    x_sel_p = pl.kernel(
        functools.partial(_sc_gather_x, b_per_w=b_per_w),
        out_type=jax.ShapeDtypeStruct((npad, f_in), jnp.float32),
        mesh=mesh,
        scratch_types=[
            pltpu.VMEM((b_per_w,), jnp.int32),
            pltpu.VMEM((b_per_w, f_in), jnp.float32),
            pltpu.SemaphoreType.DMA,
        ],
    )(idx_pad, x)
    x_sel = x_sel_p[:nn_count]

    bm = _pick_bm(n, 400)
    grid_n = n // bm
    w2_bf = W_enc2.astype(jnp.bfloat16)
    b2 = pl.pallas_call(
        functools.partial(_pass1_body, h1=h1, bm=bm, n=n, nn_count=nn_count),
        grid=(grid_n,),
        in_specs=[
            pl.BlockSpec((1, nn_count), lambda i: (0, 0)),
            pl.BlockSpec((nn_count, f_in), lambda i: (0, 0)),
            pl.BlockSpec((nn_count, f_in), lambda i: (0, 0)),
            pl.BlockSpec((n, f_in), lambda i: (0, 0)),
            pl.BlockSpec((f_in, h1), lambda i: (0, 0)),
            pl.BlockSpec((h1, h2), lambda i: (0, 0)),
            pl.BlockSpec((bm, n), lambda i: (i, 0)),
        ],
        out_specs=pl.BlockSpec((bm, 2 * h2), lambda i: (i, 0)),
        out_shape=jax.ShapeDtypeStruct((n, 2 * h2), jnp.bfloat16),
        scratch_shapes=[pltpu.VMEM((n, 2 * h1), jnp.bfloat16)],
        compiler_params=pltpu.CompilerParams(
            dimension_semantics=("arbitrary",)),
    )(nn2d, x_sel, noise_n, x, W_enc1, w2_bf, adj)

    z, emb, rep = pl.pallas_call(
        functools.partial(_pass2_body, h2=h2),
        grid=(grid_n,),
        in_specs=[
            pl.BlockSpec((bm, n), lambda i: (i, 0)),
            pl.BlockSpec((n, 2 * h2), lambda i: (0, 0)),
            pl.BlockSpec((h2, h2), lambda i: (0, 0)),
        ],
        out_specs=[
            pl.BlockSpec((bm, h2), lambda i: (i, 0)),
            pl.BlockSpec((bm, h2), lambda i: (i, 0)),
            pl.BlockSpec((bm, h2), lambda i: (i, 0)),
        ],
        out_shape=[
            jax.ShapeDtypeStruct((n, h2), jnp.float32),
            jax.ShapeDtypeStruct((n, h2), jnp.float32),
            jax.ShapeDtypeStruct((n, h2), jnp.float32),
        ],
        compiler_params=pltpu.CompilerParams(
            dimension_semantics=("arbitrary",)),
    )(adj, b2, W_e2d)

    bm5 = _pick_bm(nn_count, 200)
    x_rec = pl.pallas_call(
        functools.partial(_dec_body, bm5=bm5),
        grid_spec=pltpu.PrefetchScalarGridSpec(
            num_scalar_prefetch=1,
            grid=(nn_count // bm5,),
            in_specs=[
                pl.BlockSpec(memory_space=pl.ANY),
                pl.BlockSpec((n, h2), lambda i, idx_ref: (0, 0)),
                pl.BlockSpec((h2, f_in), lambda i, idx_ref: (0, 0)),
            ],
            out_specs=pl.BlockSpec((bm5, f_in), lambda i, idx_ref: (i, 0)),
            scratch_shapes=[
                pltpu.VMEM((bm5, n), jnp.float32),
                pltpu.SemaphoreType.DMA,
            ],
        ),
        out_shape=jax.ShapeDtypeStruct((nn_count, f_in), jnp.float32),
    )(idx, adj, rep, W_dec)

    return (x_sel, x_rec, emb, rep, z)


# speedup vs baseline: 1.1316x; 1.0307x over previous
import functools

import jax
import jax.numpy as jnp
from jax import lax
from jax.experimental import pallas as pl
from jax.experimental.pallas import tpu as pltpu
from jax.experimental.pallas import tpu_sc as plsc

EPS = 0.1


def _pick_bm(n, target):
    for bm in range(min(n, target), 0, -1):
        if n % bm == 0 and (bm % 8 == 0 or bm == n):
            return bm
    return n


def _sc_gather_x(idx_hbm, x_hbm, out_x, idx_v, xrows_v, sem, *, b_per_w):
    nc = plsc.get_sparse_core_info().num_cores
    wid = lax.axis_index("s") * nc + lax.axis_index("c")
    base = wid * b_per_w
    pltpu.sync_copy(idx_hbm.at[pl.ds(base, b_per_w)], idx_v)
    pltpu.async_copy(x_hbm.at[idx_v], xrows_v, sem).wait()
    pltpu.sync_copy(xrows_v, out_x.at[pl.ds(base, b_per_w)])


def _pass1_body(nn_ref, xsel_ref, noisen_ref, x_ref, w1_ref, w2_ref, adj_ref,
                out_ref, b1_scr, *, h1, bm, n, nn_count):
    i = pl.program_id(0)

    @pl.when(i == 0)
    def _():
        w1b = w1_ref[...].astype(jnp.bfloat16)
        delta = jnp.sign(xsel_ref[...]) * noisen_ref[...]
        dw1 = jnp.dot(delta.astype(jnp.bfloat16), w1b,
                      preferred_element_type=jnp.float32).astype(jnp.bfloat16)
        for c in range(n // bm):
            rows = c * bm + jax.lax.broadcasted_iota(
                jnp.int32, (bm, nn_count), 0)
            oh = (rows == nn_ref[...]).astype(jnp.bfloat16)
            d1 = jnp.dot(oh, dw1, preferred_element_type=jnp.float32)
            p = jnp.dot(x_ref[pl.ds(c * bm, bm), :].astype(jnp.bfloat16), w1b,
                        preferred_element_type=jnp.float32)
            b1_scr[pl.ds(c * bm, bm), :] = jnp.concatenate(
                [p + d1, p], axis=1).astype(jnp.bfloat16)

    adj_bf = adj_ref[...].astype(jnp.bfloat16)
    h = jnp.maximum(
        jnp.dot(adj_bf, b1_scr[...], preferred_element_type=jnp.float32), 0.0)
    hb = h.astype(jnp.bfloat16)
    w2 = w2_ref[...]
    b2a = jnp.dot(hb[:, :h1], w2, preferred_element_type=jnp.float32)
    b2b = jnp.dot(hb[:, h1:], w2, preferred_element_type=jnp.float32)
    out_ref[...] = jnp.concatenate([b2a, b2b], axis=1).astype(jnp.bfloat16)


def _pass2_body(adj_ref, b2_ref, we2d_ref, z_ref, emb_ref, rep_ref, *, h2):
    adj_bf = adj_ref[...].astype(jnp.bfloat16)
    ze = jnp.dot(adj_bf, b2_ref[...], preferred_element_type=jnp.float32)
    z = ze[:, :h2]
    z_ref[...] = z
    emb_ref[...] = ze[:, h2:]
    rep_ref[...] = jnp.dot(z, we2d_ref[...], preferred_element_type=jnp.float32)


def _dec_body(idx_ref, adj_ref, rep_ref, wdec_ref, out_ref, buf, sems,
              *, bm5, nblk):
    i = pl.program_id(0)

    def issue(block, slot):
        for j in range(bm5):
            pltpu.make_async_copy(
                adj_ref.at[pl.ds(idx_ref[block * bm5 + j], 1), :],
                buf.at[slot, pl.ds(j, 1), :], sems.at[slot]).start()

    def finish(slot):
        for j in range(bm5):
            pltpu.make_async_copy(
                adj_ref.at[pl.ds(idx_ref[i * bm5 + j], 1), :],
                buf.at[slot, pl.ds(j, 1), :], sems.at[slot]).wait()
        rep_bf = rep_ref[...].astype(jnp.bfloat16)
        t = jnp.dot(buf[slot].astype(jnp.bfloat16), rep_bf,
                    preferred_element_type=jnp.float32)
        out_ref[...] = jnp.dot(t.astype(jnp.bfloat16),
                               wdec_ref[...].astype(jnp.bfloat16),
                               preferred_element_type=jnp.float32)

    @pl.when(i == 0)
    def _():
        issue(0, 0)

    @pl.when(jnp.logical_and(i + 1 < nblk, (i + 1) % 2 == 0))
    def _():
        issue(i + 1, 0)

    @pl.when(jnp.logical_and(i + 1 < nblk, (i + 1) % 2 == 1))
    def _():
        issue(i + 1, 1)

    @pl.when(i % 2 == 0)
    def _():
        finish(0)

    @pl.when(i % 2 == 1)
    def _():
        finish(1)


def kernel(adj, x, noise_nodes, W_enc1, W_enc2, W_e2d, W_dec):
    n, f_in = x.shape
    h1 = W_enc1.shape[1]
    h2 = W_enc2.shape[1]
    nn_count = noise_nodes.shape[0]
    idx = noise_nodes.astype(jnp.int32)
    nn2d = idx.reshape(1, nn_count)
    noise = jax.random.uniform(
        jax.random.key(42), (nn_count, f_in), dtype=x.dtype)
    noise_n = noise / (jnp.linalg.norm(
        noise, axis=-1, keepdims=True) + 1e-12) * EPS

    info = plsc.get_sparse_core_info()
    nw = info.num_cores * info.num_subcores
    npad = ((nn_count + 8 * nw - 1) // (8 * nw)) * (8 * nw)
    idx_pad = jnp.pad(idx, (0, npad - nn_count))
    b_per_w = npad // nw
    chunk = 8
    mesh = plsc.VectorSubcoreMesh(core_axis_name="c", subcore_axis_name="s")
    x_sel_p = pl.kernel(
        functools.partial(_sc_gather_x, b_per_w=b_per_w),
        out_type=jax.ShapeDtypeStruct((npad, f_in), jnp.float32),
        mesh=mesh,
        scratch_types=[
            pltpu.VMEM((b_per_w,), jnp.int32),
            pltpu.VMEM((b_per_w, f_in), jnp.float32),
            pltpu.SemaphoreType.DMA,
        ],
    )(idx_pad, x)
    x_sel = x_sel_p[:nn_count]

    bm = _pick_bm(n, 400)
    grid_n = n // bm
    w2_bf = W_enc2.astype(jnp.bfloat16)
    b2 = pl.pallas_call(
        functools.partial(_pass1_body, h1=h1, bm=bm, n=n, nn_count=nn_count),
        grid=(grid_n,),
        in_specs=[
            pl.BlockSpec((1, nn_count), lambda i: (0, 0)),
            pl.BlockSpec((nn_count, f_in), lambda i: (0, 0)),
            pl.BlockSpec((nn_count, f_in), lambda i: (0, 0)),
            pl.BlockSpec((n, f_in), lambda i: (0, 0)),
            pl.BlockSpec((f_in, h1), lambda i: (0, 0)),
            pl.BlockSpec((h1, h2), lambda i: (0, 0)),
            pl.BlockSpec((bm, n), lambda i: (i, 0)),
        ],
        out_specs=pl.BlockSpec((bm, 2 * h2), lambda i: (i, 0)),
        out_shape=jax.ShapeDtypeStruct((n, 2 * h2), jnp.bfloat16),
        scratch_shapes=[pltpu.VMEM((n, 2 * h1), jnp.bfloat16)],
        compiler_params=pltpu.CompilerParams(
            dimension_semantics=("arbitrary",)),
    )(nn2d, x_sel, noise_n, x, W_enc1, w2_bf, adj)

    z, emb, rep = pl.pallas_call(
        functools.partial(_pass2_body, h2=h2),
        grid=(grid_n,),
        in_specs=[
            pl.BlockSpec((bm, n), lambda i: (i, 0)),
            pl.BlockSpec((n, 2 * h2), lambda i: (0, 0)),
            pl.BlockSpec((h2, h2), lambda i: (0, 0)),
        ],
        out_specs=[
            pl.BlockSpec((bm, h2), lambda i: (i, 0)),
            pl.BlockSpec((bm, h2), lambda i: (i, 0)),
            pl.BlockSpec((bm, h2), lambda i: (i, 0)),
        ],
        out_shape=[
            jax.ShapeDtypeStruct((n, h2), jnp.float32),
            jax.ShapeDtypeStruct((n, h2), jnp.float32),
            jax.ShapeDtypeStruct((n, h2), jnp.float32),
        ],
        compiler_params=pltpu.CompilerParams(
            dimension_semantics=("arbitrary",)),
    )(adj, b2, W_e2d)

    bm5 = _pick_bm(nn_count, 200)
    nblk = nn_count // bm5
    x_rec = pl.pallas_call(
        functools.partial(_dec_body, bm5=bm5, nblk=nblk),
        grid_spec=pltpu.PrefetchScalarGridSpec(
            num_scalar_prefetch=1,
            grid=(nblk,),
            in_specs=[
                pl.BlockSpec(memory_space=pl.ANY),
                pl.BlockSpec((n, h2), lambda i, idx_ref: (0, 0)),
                pl.BlockSpec((h2, f_in), lambda i, idx_ref: (0, 0)),
            ],
            out_specs=pl.BlockSpec((bm5, f_in), lambda i, idx_ref: (i, 0)),
            scratch_shapes=[
                pltpu.VMEM((2, bm5, n), jnp.float32),
                pltpu.SemaphoreType.DMA((2,)),
            ],
        ),
        out_shape=jax.ShapeDtypeStruct((nn_count, f_in), jnp.float32),
    )(idx, adj, rep, W_dec)

    return (x_sel, x_rec, emb, rep, z)
